# baseline (device time: 44366 ns/iter reference)
import jax
import jax.numpy as jnp
from jax import lax
from jax.experimental import pallas as pl
from jax.experimental.pallas import tpu as pltpu

N_DEV = 4
SUB = 2


def kernel(x):
    m, n = x.shape
    mc = m // N_DEV
    hc = n // 2
    qc = hc // SUB

    def body(
        x_ref,
        out_ref,
        accA,
        accB,
        rA1,
        rB1,
        rA2,
        rB2,
        own_sems,
        send_sems,
        recv_sems,
    ):
        d = lax.axis_index("i")
        p1 = d ^ 1
        p2 = 3 - d

        barrier_sem = pltpu.get_barrier_semaphore()
        for nbr in [p1, p2]:
            pl.semaphore_signal(
                barrier_sem,
                inc=1,
                device_id=(nbr,),
                device_id_type=pl.DeviceIdType.MESH,
            )

        ownA = lax.rem(d + 3, N_DEV)
        keepA = jnp.where(ownA >= 2, 2, 0)
        sendA1 = 2 - keepA
        sendA2 = lax.rem(6 - d, N_DEV)
        offA2 = sendA2 - keepA
        offAo = ownA - keepA
        keepB = jnp.where(d >= 2, 2, 0)
        sendB1 = 2 - keepB
        sendB2 = p1
        offB2 = sendB2 - keepB
        offBo = d - keepB

        def colA(s):
            return pl.ds(s * qc, qc)

        def colB(s):
            return pl.ds(hc + s * qc, qc)

        def exchange(src, dst, k, s, partner):
            return pltpu.make_async_remote_copy(
                src_ref=src,
                dst_ref=dst,
                send_sem=send_sems.at[k, s],
                recv_sem=recv_sems.at[k, s],
                device_id=(partner,),
                device_id_type=pl.DeviceIdType.MESH,
            )

        a1, b1 = [], []
        for s in range(SUB):
            accA[s, pl.ds(sendA1, 2)] = (
                x_ref[pl.ds(sendA1 * mc, 2 * mc), colA(s)]
                .astype(jnp.bfloat16)
                .reshape(2, mc, qc)
            )
            if s == 0:
                pl.semaphore_wait(barrier_sem, 2)
            a1.append(exchange(accA.at[s, pl.ds(sendA1, 2)], rA1.at[s], 0, s, p1))
            a1[s].start()
        for s in range(SUB):
            accB[s, pl.ds(sendB1, 2)] = (
                x_ref[pl.ds(sendB1 * mc, 2 * mc), colB(s)]
                .astype(jnp.bfloat16)
                .reshape(2, mc, qc)
            )
            b1.append(exchange(accB.at[s, pl.ds(sendB1, 2)], rB1.at[s], 1, s, p2))
            b1[s].start()

        for s in range(SUB):
            accA[s, pl.ds(keepA, 2)] = (
                x_ref[pl.ds(keepA * mc, 2 * mc), colA(s)]
                .astype(jnp.bfloat16)
                .reshape(2, mc, qc)
            )
            accB[s, pl.ds(keepB, 2)] = (
                x_ref[pl.ds(keepB * mc, 2 * mc), colB(s)]
                .astype(jnp.bfloat16)
                .reshape(2, mc, qc)
            )

        a2, b2 = [], []
        for s in range(SUB):
            a1[s].wait()
            accA[s, pl.ds(sendA2, 1)] = (
                accA[s, pl.ds(sendA2, 1)] + rA1[s, pl.ds(offA2, 1)]
            )
            a2.append(exchange(accA.at[s, pl.ds(sendA2, 1)], rA2.at[s], 2, s, p2))
            a2[s].start()
            b1[s].wait()
            accB[s, pl.ds(sendB2, 1)] = (
                accB[s, pl.ds(sendB2, 1)] + rB1[s, pl.ds(offB2, 1)]
            )
            b2.append(exchange(accB.at[s, pl.ds(sendB2, 1)], rB2.at[s], 3, s, p1))
            b2[s].start()
            accA[s, pl.ds(ownA, 1)] = (
                accA[s, pl.ds(ownA, 1)] + rA1[s, pl.ds(offAo, 1)]
            )
            accB[s, pl.ds(d, 1)] = (
                accB[s, pl.ds(d, 1)] + rB1[s, pl.ds(offBo, 1)]
            )

        g1a, g1b, ownA_cp, ownB_cp = [], [], [], []
        for s in range(SUB):
            a2[s].wait()
            accA[s, pl.ds(ownA, 1)] = accA[s, pl.ds(ownA, 1)] + rA2[s]
            cp = pltpu.make_async_copy(
                accA.at[s, ownA],
                out_ref.at[pl.ds(ownA * mc, mc), colA(s)],
                own_sems.at[0, s],
            )
            cp.start()
            ownA_cp.append(cp)
            g1a.append(
                exchange(
                    accA.at[s, ownA],
                    out_ref.at[pl.ds(ownA * mc, mc), colA(s)],
                    4,
                    s,
                    p2,
                )
            )
            g1a[s].start()
            b2[s].wait()
            accB[s, pl.ds(d, 1)] = accB[s, pl.ds(d, 1)] + rB2[s]
            cp = pltpu.make_async_copy(
                accB.at[s, d],
                out_ref.at[pl.ds(d * mc, mc), colB(s)],
                own_sems.at[1, s],
            )
            cp.start()
            ownB_cp.append(cp)
            g1b.append(
                exchange(
                    accB.at[s, d],
                    out_ref.at[pl.ds(d * mc, mc), colB(s)],
                    5,
                    s,
                    p1,
                )
            )
            g1b[s].start()

        g2a, g2b = [], []
        for s in range(SUB):
            g1a[s].wait()
            ownA_cp[s].wait()
            g2a.append(
                exchange(
                    out_ref.at[pl.ds(keepA * mc, 2 * mc), colA(s)],
                    out_ref.at[pl.ds(keepA * mc, 2 * mc), colA(s)],
                    6,
                    s,
                    p1,
                )
            )
            g2a[s].start()
            g1b[s].wait()
            ownB_cp[s].wait()
            g2b.append(
                exchange(
                    out_ref.at[pl.ds(keepB * mc, 2 * mc), colB(s)],
                    out_ref.at[pl.ds(keepB * mc, 2 * mc), colB(s)],
                    7,
                    s,
                    p2,
                )
            )
            g2b[s].start()

        for s in range(SUB):
            g2a[s].wait()
            g2b[s].wait()

    return pl.pallas_call(
        body,
        out_shape=jax.ShapeDtypeStruct((m, n), jnp.bfloat16),
        in_specs=[pl.BlockSpec(memory_space=pltpu.VMEM)],
        out_specs=pl.BlockSpec(memory_space=pl.ANY),
        scratch_shapes=[
            pltpu.VMEM((SUB, N_DEV, mc, qc), jnp.bfloat16),
            pltpu.VMEM((SUB, N_DEV, mc, qc), jnp.bfloat16),
            pltpu.VMEM((SUB, 2, mc, qc), jnp.bfloat16),
            pltpu.VMEM((SUB, 2, mc, qc), jnp.bfloat16),
            pltpu.VMEM((SUB, 1, mc, qc), jnp.bfloat16),
            pltpu.VMEM((SUB, 1, mc, qc), jnp.bfloat16),
            pltpu.SemaphoreType.DMA((2, SUB)),
            pltpu.SemaphoreType.DMA((8, SUB)),
            pltpu.SemaphoreType.DMA((8, SUB)),
        ],
        compiler_params=pltpu.CompilerParams(collective_id=0),
    )(x)


# device time: 44312 ns/iter; 1.0012x vs baseline; 1.0012x over previous
import jax
import jax.numpy as jnp
from jax import lax
from jax.experimental import pallas as pl
from jax.experimental.pallas import tpu as pltpu

N_DEV = 4
SUB = 2


def kernel(x):
    m, n = x.shape
    mc = m // N_DEV
    hc = n // 2
    qc = hc // SUB

    def body(
        x_ref,
        out_ref,
        accA,
        accB,
        rA1,
        rB1,
        rA2,
        rB2,
        send_sems,
        recv_sems,
    ):
        d = lax.axis_index("i")
        p1 = d ^ 1
        p2 = 3 - d

        barrier_sem = pltpu.get_barrier_semaphore()
        for nbr in [p1, p2]:
            pl.semaphore_signal(
                barrier_sem,
                inc=1,
                device_id=(nbr,),
                device_id_type=pl.DeviceIdType.MESH,
            )

        ownA = lax.rem(d + 3, N_DEV)
        keepA = jnp.where(ownA >= 2, 2, 0)
        sendA1 = 2 - keepA
        sendA2 = lax.rem(6 - d, N_DEV)
        offA2 = sendA2 - keepA
        offAo = ownA - keepA
        keepB = jnp.where(d >= 2, 2, 0)
        sendB1 = 2 - keepB
        sendB2 = p1
        offB2 = sendB2 - keepB
        offBo = d - keepB

        def colA(s):
            return pl.ds(s * qc, qc)

        def colB(s):
            return pl.ds(hc + s * qc, qc)

        def exchange(src, dst, k, s, partner):
            return pltpu.make_async_remote_copy(
                src_ref=src,
                dst_ref=dst,
                send_sem=send_sems.at[k, s],
                recv_sem=recv_sems.at[k, s],
                device_id=(partner,),
                device_id_type=pl.DeviceIdType.MESH,
            )

        a1, b1 = [], []
        for s in range(SUB):
            accA[s, pl.ds(sendA1, 2)] = (
                x_ref[pl.ds(sendA1 * mc, 2 * mc), colA(s)]
                .astype(jnp.bfloat16)
                .reshape(2, mc, qc)
            )
            if s == 0:
                pl.semaphore_wait(barrier_sem, 2)
            a1.append(exchange(accA.at[s, pl.ds(sendA1, 2)], rA1.at[s], 0, s, p1))
            a1[s].start()
        for s in range(SUB):
            accB[s, pl.ds(sendB1, 2)] = (
                x_ref[pl.ds(sendB1 * mc, 2 * mc), colB(s)]
                .astype(jnp.bfloat16)
                .reshape(2, mc, qc)
            )
            b1.append(exchange(accB.at[s, pl.ds(sendB1, 2)], rB1.at[s], 1, s, p2))
            b1[s].start()

        for s in range(SUB):
            accA[s, pl.ds(keepA, 2)] = (
                x_ref[pl.ds(keepA * mc, 2 * mc), colA(s)]
                .astype(jnp.bfloat16)
                .reshape(2, mc, qc)
            )
            accB[s, pl.ds(keepB, 2)] = (
                x_ref[pl.ds(keepB * mc, 2 * mc), colB(s)]
                .astype(jnp.bfloat16)
                .reshape(2, mc, qc)
            )

        a2, b2 = [], []
        for s in range(SUB):
            a1[s].wait()
            accA[s, pl.ds(sendA2, 1)] = (
                accA[s, pl.ds(sendA2, 1)] + rA1[s, pl.ds(offA2, 1)]
            )
            a2.append(exchange(accA.at[s, pl.ds(sendA2, 1)], rA2.at[s], 2, s, p2))
            a2[s].start()
            b1[s].wait()
            accB[s, pl.ds(sendB2, 1)] = (
                accB[s, pl.ds(sendB2, 1)] + rB1[s, pl.ds(offB2, 1)]
            )
            b2.append(exchange(accB.at[s, pl.ds(sendB2, 1)], rB2.at[s], 3, s, p1))
            b2[s].start()
            accA[s, pl.ds(ownA, 1)] = (
                accA[s, pl.ds(ownA, 1)] + rA1[s, pl.ds(offAo, 1)]
            )
            accB[s, pl.ds(d, 1)] = (
                accB[s, pl.ds(d, 1)] + rB1[s, pl.ds(offBo, 1)]
            )

        g1a, g1b = [], []
        for s in range(SUB):
            a2[s].wait()
            out_ref[pl.ds(ownA * mc, mc), colA(s)] = (
                accA[s, pl.ds(ownA, 1)] + rA2[s]
            ).reshape(mc, qc)
            g1a.append(
                exchange(
                    out_ref.at[pl.ds(ownA * mc, mc), colA(s)],
                    out_ref.at[pl.ds(ownA * mc, mc), colA(s)],
                    4,
                    s,
                    p2,
                )
            )
            g1a[s].start()
            b2[s].wait()
            out_ref[pl.ds(d * mc, mc), colB(s)] = (
                accB[s, pl.ds(d, 1)] + rB2[s]
            ).reshape(mc, qc)
            g1b.append(
                exchange(
                    out_ref.at[pl.ds(d * mc, mc), colB(s)],
                    out_ref.at[pl.ds(d * mc, mc), colB(s)],
                    5,
                    s,
                    p1,
                )
            )
            g1b[s].start()

        g2a, g2b = [], []
        for s in range(SUB):
            g1a[s].wait()
            g2a.append(
                exchange(
                    out_ref.at[pl.ds(keepA * mc, 2 * mc), colA(s)],
                    out_ref.at[pl.ds(keepA * mc, 2 * mc), colA(s)],
                    6,
                    s,
                    p1,
                )
            )
            g2a[s].start()
            g1b[s].wait()
            g2b.append(
                exchange(
                    out_ref.at[pl.ds(keepB * mc, 2 * mc), colB(s)],
                    out_ref.at[pl.ds(keepB * mc, 2 * mc), colB(s)],
                    7,
                    s,
                    p2,
                )
            )
            g2b[s].start()

        for s in range(SUB):
            g2a[s].wait()
            g2b[s].wait()

    return pl.pallas_call(
        body,
        out_shape=jax.ShapeDtypeStruct((m, n), jnp.bfloat16),
        in_specs=[pl.BlockSpec(memory_space=pltpu.VMEM)],
        out_specs=pl.BlockSpec(memory_space=pltpu.VMEM),
        scratch_shapes=[
            pltpu.VMEM((SUB, N_DEV, mc, qc), jnp.bfloat16),
            pltpu.VMEM((SUB, N_DEV, mc, qc), jnp.bfloat16),
            pltpu.VMEM((SUB, 2, mc, qc), jnp.bfloat16),
            pltpu.VMEM((SUB, 2, mc, qc), jnp.bfloat16),
            pltpu.VMEM((SUB, 1, mc, qc), jnp.bfloat16),
            pltpu.VMEM((SUB, 1, mc, qc), jnp.bfloat16),
            pltpu.SemaphoreType.DMA((8, SUB)),
            pltpu.SemaphoreType.DMA((8, SUB)),
        ],
        compiler_params=pltpu.CompilerParams(collective_id=0),
    )(x)


# device time: 44208 ns/iter; 1.0036x vs baseline; 1.0024x over previous
import jax
import jax.numpy as jnp
from jax import lax
from jax.experimental import pallas as pl
from jax.experimental.pallas import tpu as pltpu

N_DEV = 4
SUB = 2


def kernel(x):
    m, n = x.shape
    mc = m // N_DEV
    hc = n // 2
    qc = hc // SUB

    def body(
        x_ref,
        out_ref,
        accA,
        accB,
        rA1,
        rB1,
        rA2,
        rB2,
        send_sems,
        recv_sems,
    ):
        d = lax.axis_index("i")
        p1 = d ^ 1
        p2 = 3 - d

        barrier_sem = pltpu.get_barrier_semaphore()
        for nbr in [p1, p2]:
            pl.semaphore_signal(
                barrier_sem,
                inc=1,
                device_id=(nbr,),
                device_id_type=pl.DeviceIdType.MESH,
            )

        ownA = lax.rem(d + 3, N_DEV)
        keepA = jnp.where(ownA >= 2, 2, 0)
        sendA1 = 2 - keepA
        sendA2 = lax.rem(6 - d, N_DEV)
        offA2 = sendA2 - keepA
        offAo = ownA - keepA
        keepB = jnp.where(d >= 2, 2, 0)
        sendB1 = 2 - keepB
        sendB2 = p1
        offB2 = sendB2 - keepB
        offBo = d - keepB

        def colA(s):
            return pl.ds(s * qc, qc)

        def colB(s):
            return pl.ds(hc + s * qc, qc)

        def exchange(src, dst, k, s, partner):
            return pltpu.make_async_remote_copy(
                src_ref=src,
                dst_ref=dst,
                send_sem=send_sems.at[k, s],
                recv_sem=recv_sems.at[k, s],
                device_id=(partner,),
                device_id_type=pl.DeviceIdType.MESH,
            )

        a1, b1 = [], []
        for s in range(SUB):
            accA[s, pl.ds(sendA1, 2)] = (
                x_ref[pl.ds(sendA1 * mc, 2 * mc), colA(s)]
                .astype(jnp.bfloat16)
                .reshape(2, mc, qc)
            )
            if s == 0:
                pl.semaphore_wait(barrier_sem, 2)
            a1.append(exchange(accA.at[s, pl.ds(sendA1, 2)], rA1.at[s], 0, s, p1))
            a1[s].start()
            accB[s, pl.ds(sendB1, 2)] = (
                x_ref[pl.ds(sendB1 * mc, 2 * mc), colB(s)]
                .astype(jnp.bfloat16)
                .reshape(2, mc, qc)
            )
            b1.append(exchange(accB.at[s, pl.ds(sendB1, 2)], rB1.at[s], 1, s, p2))
            b1[s].start()

        for s in range(SUB):
            accA[s, pl.ds(keepA, 2)] = (
                x_ref[pl.ds(keepA * mc, 2 * mc), colA(s)]
                .astype(jnp.bfloat16)
                .reshape(2, mc, qc)
            )
            accB[s, pl.ds(keepB, 2)] = (
                x_ref[pl.ds(keepB * mc, 2 * mc), colB(s)]
                .astype(jnp.bfloat16)
                .reshape(2, mc, qc)
            )

        a2, b2 = [], []
        for s in range(SUB):
            a1[s].wait()
            accA[s, pl.ds(sendA2, 1)] = (
                accA[s, pl.ds(sendA2, 1)] + rA1[s, pl.ds(offA2, 1)]
            )
            a2.append(exchange(accA.at[s, pl.ds(sendA2, 1)], rA2.at[s], 2, s, p2))
            a2[s].start()
            b1[s].wait()
            accB[s, pl.ds(sendB2, 1)] = (
                accB[s, pl.ds(sendB2, 1)] + rB1[s, pl.ds(offB2, 1)]
            )
            b2.append(exchange(accB.at[s, pl.ds(sendB2, 1)], rB2.at[s], 3, s, p1))
            b2[s].start()
            accA[s, pl.ds(ownA, 1)] = (
                accA[s, pl.ds(ownA, 1)] + rA1[s, pl.ds(offAo, 1)]
            )
            accB[s, pl.ds(d, 1)] = (
                accB[s, pl.ds(d, 1)] + rB1[s, pl.ds(offBo, 1)]
            )

        g1a, g1b = [], []
        for s in range(SUB):
            a2[s].wait()
            out_ref[pl.ds(ownA * mc, mc), colA(s)] = (
                accA[s, pl.ds(ownA, 1)] + rA2[s]
            ).reshape(mc, qc)
            g1a.append(
                exchange(
                    out_ref.at[pl.ds(ownA * mc, mc), colA(s)],
                    out_ref.at[pl.ds(ownA * mc, mc), colA(s)],
                    4,
                    s,
                    p2,
                )
            )
            g1a[s].start()
            b2[s].wait()
            out_ref[pl.ds(d * mc, mc), colB(s)] = (
                accB[s, pl.ds(d, 1)] + rB2[s]
            ).reshape(mc, qc)
            g1b.append(
                exchange(
                    out_ref.at[pl.ds(d * mc, mc), colB(s)],
                    out_ref.at[pl.ds(d * mc, mc), colB(s)],
                    5,
                    s,
                    p1,
                )
            )
            g1b[s].start()

        g2a, g2b = [], []
        for s in range(SUB):
            g1a[s].wait()
            g2a.append(
                exchange(
                    out_ref.at[pl.ds(keepA * mc, 2 * mc), colA(s)],
                    out_ref.at[pl.ds(keepA * mc, 2 * mc), colA(s)],
                    6,
                    s,
                    p1,
                )
            )
            g2a[s].start()
            g1b[s].wait()
            g2b.append(
                exchange(
                    out_ref.at[pl.ds(keepB * mc, 2 * mc), colB(s)],
                    out_ref.at[pl.ds(keepB * mc, 2 * mc), colB(s)],
                    7,
                    s,
                    p2,
                )
            )
            g2b[s].start()

        for s in range(SUB):
            g2a[s].wait()
            g2b[s].wait()

    return pl.pallas_call(
        body,
        out_shape=jax.ShapeDtypeStruct((m, n), jnp.bfloat16),
        in_specs=[pl.BlockSpec(memory_space=pltpu.VMEM)],
        out_specs=pl.BlockSpec(memory_space=pltpu.VMEM),
        scratch_shapes=[
            pltpu.VMEM((SUB, N_DEV, mc, qc), jnp.bfloat16),
            pltpu.VMEM((SUB, N_DEV, mc, qc), jnp.bfloat16),
            pltpu.VMEM((SUB, 2, mc, qc), jnp.bfloat16),
            pltpu.VMEM((SUB, 2, mc, qc), jnp.bfloat16),
            pltpu.VMEM((SUB, 1, mc, qc), jnp.bfloat16),
            pltpu.VMEM((SUB, 1, mc, qc), jnp.bfloat16),
            pltpu.SemaphoreType.DMA((8, SUB)),
            pltpu.SemaphoreType.DMA((8, SUB)),
        ],
        compiler_params=pltpu.CompilerParams(collective_id=0),
    )(x)
